# Initial kernel scaffold; baseline (speedup 1.0000x reference)
import jax
import jax.numpy as jnp
from jax.experimental import pallas as pl


def kernel(inputs, eps):
    raise NotImplementedError("write your pallas kernel here")


# speedup vs baseline: 138.3528x; 138.3528x over previous
import jax
import jax.numpy as jnp
from jax import lax
from jax.experimental import pallas as pl
from jax.experimental.pallas import tpu as pltpu
from jax.experimental.pallas import tpu_sc as plsc

B = 1024
L = 1024
N = 16
NC, NS, LANES = 2, 16, 16
NW = NC * NS
RPW = B // NW
NG = RPW // LANES


def _tree_sum(xs):
    while len(xs) > 1:
        xs = [xs[i] + xs[i + 1] for i in range(0, len(xs), 2)]
    return xs[0]


def _sc_body(idx_hbm, eps_hbm, out_hbm, idx_v, eps_v, out_v):
    wid = lax.axis_index("s") * NC + lax.axis_index("c")
    pltpu.sync_copy(idx_hbm.at[wid], idx_v)
    pltpu.sync_copy(eps_hbm, eps_v)

    for g in range(NG):
        def step(t, carry, g=g):
            acc = carry[0]
            P = list(carry[1:])
            srow = idx_v[pl.ds(t * RPW + g * LANES, LANES)]
            mask = srow > 0
            E0 = eps_v[pl.ds(t * (2 * N), N)]
            E1 = eps_v[pl.ds(t * (2 * N) + N, N)]
            us, ws = [], []
            for n in range(N):
                e0 = E0[n]
                e1 = E1[n]
                u = P[n] * e0
                w = P[n] * e1
                P[n] = jnp.where(mask, w, u)
                us.append(u)
                ws.append(w)
            ls0 = _tree_sum(us)
            ls1 = _tree_sum(ws)
            chosen = jnp.where(mask, ls1, ls0)
            m = jnp.maximum(ls0, ls1)
            mn = jnp.minimum(ls0, ls1)
            y = jnp.exp(2.0 * (mn - m))
            z = y / (2.0 + y)
            z2 = z * z
            log1p = 2.0 * z * (1.0 + z2 * (1.0 / 3 + z2 * (1.0 / 5 + z2 * (
                1.0 / 7 + z2 * (1.0 / 9 + z2 * (1.0 / 11))))))
            acc = acc + (chosen - (m + 0.5 * log1p))
            return (acc, *P)

        ones = jnp.ones((LANES,), jnp.float32)
        zeros = jnp.zeros((LANES,), jnp.float32)
        carry = lax.fori_loop(0, L, step, (zeros,) + (ones,) * N)
        out_v[pl.ds(g * LANES, LANES)] = carry[0]

    pltpu.sync_copy(out_v, out_hbm.at[pl.ds(wid * RPW, RPW)])


def kernel(inputs, eps):
    idx_r = jnp.transpose(inputs).reshape(L, NW, RPW).transpose(1, 0, 2)
    idx_r = idx_r.reshape(NW, L * RPW)
    eps_r = jnp.transpose(eps, (2, 0, 1)).astype(jnp.float32).reshape(L * 2 * N)
    f = pl.kernel(
        _sc_body,
        out_type=jax.ShapeDtypeStruct((B,), jnp.float32),
        mesh=plsc.VectorSubcoreMesh(core_axis_name="c", subcore_axis_name="s"),
        scratch_types=[
            pltpu.VMEM((L * RPW,), jnp.int32),
            pltpu.VMEM((L * 2 * N,), jnp.float32),
            pltpu.VMEM((RPW,), jnp.float32),
        ],
    )
    return f(idx_r, eps_r)
